# Initial kernel scaffold; baseline (speedup 1.0000x reference)
import jax
import jax.numpy as jnp
from jax.experimental import pallas as pl


def kernel(x_demand, x_measurement, edge_index_md, edge_index_mm, edge_label_index, edge_weight, user_emb, Wu_root, Wu_nbr, bu, Wm_self, Wm_nbr, bm):
    raise NotImplementedError("write your pallas kernel here")


# speedup vs baseline: 1.9713x; 1.9713x over previous
import functools

import jax
import jax.numpy as jnp
from jax import lax
from jax.experimental import pallas as pl
from jax.experimental.pallas import tpu as pltpu
from jax.experimental.pallas import tpu_sc as plsc

N_DEMAND = 2000
N_MEAS = 10000
H = 128
NC = 2
NS = 16
NW = NC * NS
L = 16

E_MD = 320000
E_MM = 320000
E_LABEL = 100000

CH = 128
EPW = 10240
E_PAD = EPW * NW
NCHUNK = EPW // CH
LPW = 3200
EL_PAD = LPW * NW
LCHUNK = LPW // CH

AGG_ROWS = 2048
SUM_ROWS = 10240
DPW = 64

_f32 = jnp.float32
_i32 = jnp.int32

_SC_PARAMS = pltpu.CompilerParams(use_tc_tiling_on_sc=False,
                                  needs_layout_passes=False)
_SC_MESH = plsc.VectorSubcoreMesh(core_axis_name="c", subcore_axis_name="s")


def _zeros16():
    return jnp.zeros((L,), _f32)


def _zero_rows(rows):
    def zb(i, _):
        for j in range(H // L):
            rows[i, pl.ds(j * L, L)] = _zeros16()
        return 0
    lax.fori_loop(0, CH, zb, 0)


def _sc_md_body(x_aug, src_md, dst_md, w_md, xdem, uemb,
                aggd_out, xd_out,
                aggd_sh, sidx, didx, wbuf, rows, dem_idx, dem_rows, sem):
    c = lax.axis_index("c")
    s = lax.axis_index("s")
    tid = c * NS + s

    _zero_rows(rows)
    pltpu.sync_copy(rows, aggd_sh.at[pl.ds(s * (AGG_ROWS // NS), CH)])
    plsc.subcore_barrier()

    base_e = tid * EPW

    def md_chunk(k, _):
        off = base_e + k * CH
        pltpu.sync_copy(src_md.at[pl.ds(off, CH)], sidx)
        pltpu.sync_copy(dst_md.at[pl.ds(off, CH)], didx)
        pltpu.sync_copy(w_md.at[pl.ds(off, CH)], wbuf)
        pltpu.async_copy(x_aug.at[sidx], rows, sem).wait()

        def scale(g, _):
            wv16 = wbuf[pl.ds(g * L, L)]
            for e2 in range(L):
                wsp = jnp.broadcast_to(wv16[e2], (L,))
                row = g * L + e2
                for j in range(H // L):
                    rows[row, pl.ds(j * L, L)] = rows[row, pl.ds(j * L, L)] * wsp
            return 0
        lax.fori_loop(0, CH // L, scale, 0)

        pltpu.sync_copy(rows, aggd_sh.at[didx], add=True)
        return 0
    lax.fori_loop(0, NCHUNK, md_chunk, 0)

    doff = tid * DPW
    pltpu.sync_copy(xdem.at[pl.ds(doff, DPW)], dem_idx)
    pltpu.async_copy(uemb.at[dem_idx], dem_rows, sem).wait()
    pltpu.sync_copy(dem_rows, xd_out.at[pl.ds(doff, DPW)])

    plsc.subcore_barrier()
    pltpu.sync_copy(aggd_sh.at[pl.ds(s * (AGG_ROWS // NS), AGG_ROWS // NS)],
                    aggd_out.at[c, pl.ds(s * (AGG_ROWS // NS), AGG_ROWS // NS)])


_sc_md = functools.partial(
    pl.kernel,
    _sc_md_body,
    out_type=(
        jax.ShapeDtypeStruct((NC, AGG_ROWS, H), _f32),
        jax.ShapeDtypeStruct((NW * DPW, H), _f32),
    ),
    mesh=_SC_MESH,
    compiler_params=_SC_PARAMS,
    scratch_types=[
        pltpu.VMEM_SHARED((AGG_ROWS, H), _f32),
        pltpu.VMEM((CH,), _i32),
        pltpu.VMEM((CH,), _i32),
        pltpu.VMEM((CH,), _f32),
        pltpu.VMEM((CH, H), _f32),
        pltpu.VMEM((DPW,), _i32),
        pltpu.VMEM((DPW, H), _f32),
        pltpu.SemaphoreType.DMA,
    ],
)


def _sc_mm_body(x_aug, s2, d2,
                summ_out, cnt_out,
                summ_sh, cnt_sh, sidx, didx, rows, ones_buf, czbuf, sem):
    c = lax.axis_index("c")
    s = lax.axis_index("s")
    tid = c * NS + s

    _zero_rows(rows)
    for t in range(SUM_ROWS // NS // CH):
        pltpu.sync_copy(rows, summ_sh.at[pl.ds(s * (SUM_ROWS // NS) + t * CH, CH)])

    def zc(i, _):
        czbuf[i, :] = _zeros16()
        ones_buf[i, :] = jnp.ones((L,), _f32)
        return 0
    lax.fori_loop(0, CH, zc, 0)
    for t in range(SUM_ROWS // NS // CH):
        pltpu.sync_copy(czbuf, cnt_sh.at[pl.ds(s * (SUM_ROWS // NS) + t * CH, CH)])
    plsc.subcore_barrier()

    base_e = tid * EPW

    def mm_chunk(k, _):
        off = base_e + k * CH
        pltpu.sync_copy(s2.at[pl.ds(off, CH)], sidx)
        pltpu.sync_copy(d2.at[pl.ds(off, CH)], didx)
        pltpu.async_copy(x_aug.at[sidx], rows, sem).wait()
        pltpu.sync_copy(rows, summ_sh.at[didx], add=True)
        pltpu.sync_copy(ones_buf, cnt_sh.at[didx], add=True)
        return 0
    lax.fori_loop(0, NCHUNK, mm_chunk, 0)

    plsc.subcore_barrier()
    pltpu.sync_copy(summ_sh.at[pl.ds(s * (SUM_ROWS // NS), SUM_ROWS // NS)],
                    summ_out.at[c, pl.ds(s * (SUM_ROWS // NS), SUM_ROWS // NS)])
    pltpu.sync_copy(cnt_sh.at[pl.ds(s * (SUM_ROWS // NS), SUM_ROWS // NS)],
                    cnt_out.at[c, pl.ds(s * (SUM_ROWS // NS), SUM_ROWS // NS)])


_sc_mm = functools.partial(
    pl.kernel,
    _sc_mm_body,
    out_type=(
        jax.ShapeDtypeStruct((NC, SUM_ROWS, H), _f32),
        jax.ShapeDtypeStruct((NC, SUM_ROWS, L), _f32),
    ),
    mesh=_SC_MESH,
    compiler_params=_SC_PARAMS,
    scratch_types=[
        pltpu.VMEM_SHARED((SUM_ROWS, H), _f32),
        pltpu.VMEM_SHARED((SUM_ROWS, L), _f32),
        pltpu.VMEM((CH,), _i32),
        pltpu.VMEM((CH,), _i32),
        pltpu.VMEM((CH, H), _f32),
        pltpu.VMEM((CH, L), _f32),
        pltpu.VMEM((CH, L), _f32),
        pltpu.SemaphoreType.DMA,
    ],
)


def _sc_decoder_body(zd, zm, labd, labm, out,
                     di, mi, zdr, zmr, obuf, semd, semm):
    c = lax.axis_index("c")
    s = lax.axis_index("s")
    tid = c * NS + s
    base = tid * LPW

    def chunk(k, _):
        off = base + k * CH
        pltpu.sync_copy(labd.at[pl.ds(off, CH)], di)
        pltpu.sync_copy(labm.at[pl.ds(off, CH)], mi)
        cd = pltpu.async_copy(zd.at[di], zdr, semd)
        cm = pltpu.async_copy(zm.at[mi], zmr, semm)
        cd.wait()
        cm.wait()

        lanes = lax.iota(_i32, L)

        def dot_group(g, _):
            vec = _zeros16()
            for e2 in range(L):
                e = g * L + e2
                acc = _zeros16()
                for j in range(H // L):
                    acc = acc + zdr[e, pl.ds(j * L, L)] * zmr[e, pl.ds(j * L, L)]
                s_ = jnp.sum(acc)
                vec = jnp.where(lanes == e2, jnp.broadcast_to(s_, (L,)), vec)
            obuf[pl.ds(g * L, L)] = vec
            return 0
        lax.fori_loop(0, CH // L, dot_group, 0)
        pltpu.sync_copy(obuf, out.at[pl.ds(off, CH)])
        return 0
    lax.fori_loop(0, LCHUNK, chunk, 0)


_sc_decoder = functools.partial(
    pl.kernel,
    _sc_decoder_body,
    out_type=jax.ShapeDtypeStruct((EL_PAD,), _f32),
    mesh=_SC_MESH,
    compiler_params=_SC_PARAMS,
    scratch_types=[
        pltpu.VMEM((CH,), _i32),
        pltpu.VMEM((CH,), _i32),
        pltpu.VMEM((CH, H), _f32),
        pltpu.VMEM((CH, H), _f32),
        pltpu.VMEM((CH,), _f32),
        pltpu.SemaphoreType.DMA,
        pltpu.SemaphoreType.DMA,
    ],
)


def _tc_demand_body(xd_ref, agg_ref, wr_ref, wn_ref, bu_ref, out_ref):
    a = agg_ref[0] + agg_ref[1]
    z = (jnp.dot(xd_ref[...], wr_ref[...], preferred_element_type=_f32)
         + jnp.dot(a, wn_ref[...], preferred_element_type=_f32)
         + bu_ref[...])
    out_ref[...] = jnp.maximum(z, 0.0)


def _tc_meas_body(x_ref, summ_ref, cnt_ref, ws_ref, wn_ref, bm_ref, out_ref):
    ssum = summ_ref[0] + summ_ref[1]
    cnt = cnt_ref[0] + cnt_ref[1]
    c0 = jnp.maximum(cnt[:, 0:1], 1.0)
    mean = ssum / c0
    z = (jnp.dot(x_ref[...], ws_ref[...], preferred_element_type=_f32)
         + jnp.dot(mean, wn_ref[...], preferred_element_type=_f32)
         + bm_ref[...])
    out_ref[...] = jnp.maximum(z, 0.0)


def kernel(x_demand, x_measurement, edge_index_md, edge_index_mm,
           edge_label_index, edge_weight, user_emb, Wu_root, Wu_nbr, bu,
           Wm_self, Wm_nbr, bm):
    i32 = _i32
    src_md = edge_index_md[0].astype(i32)
    dst_md = edge_index_md[1].astype(i32)
    s2 = edge_index_mm[0].astype(i32)
    d2 = edge_index_mm[1].astype(i32)
    lab_d = edge_label_index[0].astype(i32)
    lab_m = edge_label_index[1].astype(i32)

    x_aug = jnp.concatenate([x_measurement, jnp.zeros((1, H), _f32)], axis=0)

    npad = E_PAD - E_MD
    src_md = jnp.concatenate([src_md, jnp.full((npad,), N_MEAS, i32)])
    dst_md = jnp.concatenate([dst_md, jnp.full((npad,), N_DEMAND, i32)])
    w_md = jnp.concatenate([edge_weight, jnp.zeros((npad,), _f32)])
    s2 = jnp.concatenate([s2, jnp.full((npad,), N_MEAS, i32)])
    d2 = jnp.concatenate([d2, jnp.full((npad,), N_MEAS, i32)])

    xdem = jnp.concatenate(
        [x_demand.astype(i32), jnp.zeros((NW * DPW - N_DEMAND,), i32)])

    aggd_p, xd = _sc_md()(x_aug, src_md, dst_md, w_md, xdem, user_emb)
    summ_p, cnt_p = _sc_mm()(x_aug, s2, d2)

    z_demand = pl.pallas_call(
        _tc_demand_body,
        grid=(AGG_ROWS // 256,),
        in_specs=[
            pl.BlockSpec((256, H), lambda i: (i, 0)),
            pl.BlockSpec((NC, 256, H), lambda i: (0, i, 0)),
            pl.BlockSpec((H, H), lambda i: (0, 0)),
            pl.BlockSpec((H, H), lambda i: (0, 0)),
            pl.BlockSpec((1, H), lambda i: (0, 0)),
        ],
        out_specs=pl.BlockSpec((256, H), lambda i: (i, 0)),
        out_shape=jax.ShapeDtypeStruct((AGG_ROWS, H), _f32),
    )(xd, aggd_p, Wu_root, Wu_nbr, bu.reshape(1, H))

    z_meas = pl.pallas_call(
        _tc_meas_body,
        grid=(N_MEAS // 400,),
        in_specs=[
            pl.BlockSpec((400, H), lambda i: (i, 0)),
            pl.BlockSpec((NC, 400, H), lambda i: (0, i, 0)),
            pl.BlockSpec((NC, 400, L), lambda i: (0, i, 0)),
            pl.BlockSpec((H, H), lambda i: (0, 0)),
            pl.BlockSpec((H, H), lambda i: (0, 0)),
            pl.BlockSpec((1, H), lambda i: (0, 0)),
        ],
        out_specs=pl.BlockSpec((400, H), lambda i: (i, 0)),
        out_shape=jax.ShapeDtypeStruct((N_MEAS, H), _f32),
    )(x_measurement, summ_p, cnt_p, Wm_self, Wm_nbr, bm.reshape(1, H))

    lab_d = jnp.concatenate([lab_d, jnp.zeros((EL_PAD - E_LABEL,), i32)])
    lab_m = jnp.concatenate([lab_m, jnp.zeros((EL_PAD - E_LABEL,), i32)])

    out = _sc_decoder()(z_demand, z_meas, lab_d, lab_m)
    return out[:E_LABEL]
